# Initial kernel scaffold; baseline (speedup 1.0000x reference)
import jax
import jax.numpy as jnp
from jax.experimental import pallas as pl


def kernel(x, codebook):
    raise NotImplementedError("write your pallas kernel here")


# speedup vs baseline: 1.7454x; 1.7454x over previous
import jax
import jax.numpy as jnp
from jax import lax
from jax.experimental import pallas as pl

_PROTO_NUM = 1024
_PROTO_DIM = 256
_BLOCK_ROWS = 1024


def _vq_block(x_ref, cb_ref, proto_ref, resid_ref, loss_ref):
    x = x_ref[...]
    cb = cb_ref[...]
    xn = jnp.sum(x * x, axis=1, keepdims=True)
    cn = jnp.sum(cb * cb, axis=1)
    cross = lax.dot_general(
        x, cb, (((1,), (1,)), ((), ())), preferred_element_type=jnp.float32
    )
    dist = xn + cn[None, :] - 2.0 * cross
    idx = jnp.argmin(dist, axis=1)
    oh = (
        lax.broadcasted_iota(jnp.int32, (x.shape[0], _PROTO_NUM), 1)
        == idx[:, None]
    ).astype(jnp.float32)
    proto = lax.dot_general(
        oh, cb, (((1,), (0,)), ((), ())), preferred_element_type=jnp.float32
    )
    resid = x - proto
    proto_ref[...] = proto
    resid_ref[...] = resid

    @pl.when(pl.program_id(0) == 0)
    def _init():
        loss_ref[...] = jnp.zeros_like(loss_ref)

    loss_ref[...] += jnp.sum(resid * resid).reshape(1, 1)

    @pl.when(pl.program_id(0) == pl.num_programs(0) - 1)
    def _finish():
        total = loss_ref[...]
        m = total / (pl.num_programs(0) * _BLOCK_ROWS * _PROTO_DIM)
        loss_ref[...] = m + 0.25 * m


def kernel(x, codebook):
    x_shape = x.shape
    xf = x.reshape(-1, _PROTO_DIM)
    n_rows = xf.shape[0]
    grid = n_rows // _BLOCK_ROWS

    proto, resid, loss = pl.pallas_call(
        _vq_block,
        grid=(grid,),
        in_specs=[
            pl.BlockSpec((_BLOCK_ROWS, _PROTO_DIM), lambda i: (i, 0)),
            pl.BlockSpec((_PROTO_NUM, _PROTO_DIM), lambda i: (0, 0)),
        ],
        out_specs=[
            pl.BlockSpec((_BLOCK_ROWS, _PROTO_DIM), lambda i: (i, 0)),
            pl.BlockSpec((_BLOCK_ROWS, _PROTO_DIM), lambda i: (i, 0)),
            pl.BlockSpec((1, 1), lambda i: (0, 0)),
        ],
        out_shape=[
            jax.ShapeDtypeStruct((n_rows, _PROTO_DIM), jnp.float32),
            jax.ShapeDtypeStruct((n_rows, _PROTO_DIM), jnp.float32),
            jax.ShapeDtypeStruct((1, 1), jnp.float32),
        ],
    )(xf, codebook)

    return (
        proto.reshape(x_shape),
        resid.reshape(x_shape),
        loss.reshape(()),
    )
